# baseline (speedup 1.0000x reference)
import functools

import jax
import jax.numpy as jnp
from jax import lax
from jax.experimental import pallas as pl
from jax.experimental.pallas import tpu as pltpu
from jax.experimental.pallas import tpu_sc as plsc

D = 768
E = 8
F = 3072
T = 2048
LANES = 128
BT = 64
TP = T + E * BT
NTP = TP // BT
LBW = 0.01

NC = 2
NS = 16
NW = NC * NS
BPW = T // NW


def _router_body(x_ref, wr_ref, dest_ref, wrep_ref, offs_ref, aux_ref):
    x = x_ref[...]
    wr = wr_ref[...]
    logits = jnp.dot(x, wr, preferred_element_type=jnp.float32)
    m = jnp.max(logits, axis=1, keepdims=True)
    u = jnp.exp(logits - m)
    s = jnp.sum(u, axis=1, keepdims=True)
    probs = u / s
    wmax = jnp.max(probs, axis=1, keepdims=True)
    lane8 = lax.broadcasted_iota(jnp.int32, (T, E), 1)
    idx = jnp.min(jnp.where(probs == wmax, lane8, E), axis=1,
                  keepdims=True)
    lane = lax.broadcasted_iota(jnp.int32, (T, LANES), 1)
    onehot = (lane == idx).astype(jnp.float32)

    c = onehot
    sh = 1
    while sh < T:
        c = c + jnp.concatenate(
            [jnp.zeros((sh, LANES), jnp.float32), c[:-sh]], axis=0)
        sh *= 2
    counts = c[T - 1:T, :]

    i2 = lax.broadcasted_iota(jnp.int32, (LANES, LANES), 0)
    j2 = lax.broadcasted_iota(jnp.int32, (LANES, LANES), 1)
    tri = (i2 < j2).astype(jnp.float32)
    padded = jnp.floor((counts + (BT - 1)) / BT) * BT
    counts8 = jnp.broadcast_to(padded, (8, LANES))
    offs8 = jnp.dot(counts8, tri, preferred_element_type=jnp.float32,
                    precision=lax.Precision.HIGHEST)
    offs = offs8[0:1, :]

    dest = jnp.sum(onehot * (offs + c - 1.0), axis=1, keepdims=True)
    dest_ref[...] = jnp.broadcast_to(dest, (T, LANES)).astype(jnp.int32)
    wrep_ref[...] = jnp.broadcast_to(wmax, (T, LANES))

    lane_row = lane[0:1, :]
    for i in range(16):
        oi = jnp.sum(jnp.where(lane_row == i, offs, 0.0))
        offs_ref[i] = oi.astype(jnp.int32)

    mean_p8 = jnp.sum(probs, axis=0, keepdims=True) / T
    counts8 = jnp.sum((lane8 == idx).astype(jnp.float32), axis=0,
                      keepdims=True)
    aux_ref[0] = LBW * E * jnp.sum(mean_p8 * (counts8 / T))


def _router_call(x_flat, wr):
    return pl.pallas_call(
        _router_body,
        out_shape=(
            jax.ShapeDtypeStruct((T, LANES), jnp.int32),
            jax.ShapeDtypeStruct((T, LANES), jnp.float32),
            jax.ShapeDtypeStruct((16,), jnp.int32),
            jax.ShapeDtypeStruct((1,), jnp.float32),
        ),
        out_specs=(
            pl.BlockSpec(memory_space=pltpu.MemorySpace.VMEM),
            pl.BlockSpec(memory_space=pltpu.MemorySpace.VMEM),
            pl.BlockSpec(memory_space=pltpu.SMEM),
            pl.BlockSpec(memory_space=pltpu.SMEM),
        ),
    )(x_flat, wr)


def _dispatch_call(dest1d, x_flat, wrep):
    mesh = plsc.VectorSubcoreMesh(core_axis_name="c", subcore_axis_name="s")

    def body(dest_hbm, x_hbm, w_hbm, xs_hbm, ws_hbm, idx_v, xrows_v, wrows_v,
             sem1, sem2):
        wid = lax.axis_index("s") * NC + lax.axis_index("c")
        base = wid * BPW
        pltpu.sync_copy(dest_hbm.at[pl.ds(base, BPW)], idx_v)
        pltpu.sync_copy(x_hbm.at[pl.ds(base, BPW)], xrows_v)
        pltpu.sync_copy(w_hbm.at[pl.ds(base, BPW)], wrows_v)
        pltpu.async_copy(xrows_v, xs_hbm.at[idx_v], sem1).wait()
        pltpu.async_copy(wrows_v, ws_hbm.at[idx_v], sem2).wait()

    f = pl.kernel(
        body,
        out_type=(
            jax.ShapeDtypeStruct((TP, D), jnp.float32),
            jax.ShapeDtypeStruct((TP, LANES), jnp.float32),
        ),
        mesh=mesh,
        scratch_types=[
            pltpu.VMEM((BPW,), jnp.int32),
            pltpu.VMEM((BPW, D), jnp.float32),
            pltpu.VMEM((BPW, LANES), jnp.float32),
            pltpu.SemaphoreType.DMA,
            pltpu.SemaphoreType.DMA,
        ],
    )
    return f(dest1d, x_flat, wrep)


def _combine_call(dest1d, y_sorted):
    mesh = plsc.VectorSubcoreMesh(core_axis_name="c", subcore_axis_name="s")

    def body(dest_hbm, y_hbm, out_hbm, idx_v, rows_v, sem):
        wid = lax.axis_index("s") * NC + lax.axis_index("c")
        base = wid * BPW
        pltpu.sync_copy(dest_hbm.at[pl.ds(base, BPW)], idx_v)
        pltpu.async_copy(y_hbm.at[idx_v], rows_v, sem).wait()
        pltpu.sync_copy(rows_v, out_hbm.at[pl.ds(base, BPW)])

    f = pl.kernel(
        body,
        out_type=jax.ShapeDtypeStruct((T, D), jnp.float32),
        mesh=mesh,
        scratch_types=[
            pltpu.VMEM((BPW,), jnp.int32),
            pltpu.VMEM((BPW, D), jnp.float32),
            pltpu.SemaphoreType.DMA,
        ],
    )
    return f(dest1d, y_sorted)


_SQRT_HALF = 0.7071067811865476


def _ffn_body(offs_sref, x_ref, w_ref, w1_ref, b1_ref, w2_ref, b2_ref, out_ref):
    xb = x_ref[...]
    h = jnp.dot(xb, w1_ref[0], preferred_element_type=jnp.float32)
    h = h + b1_ref[0]
    g = 0.5 * h * (1.0 + lax.erf(h * _SQRT_HALF))
    yb = jnp.dot(g, w2_ref[0], preferred_element_type=jnp.float32)
    yb = yb + b2_ref[0]
    out_ref[...] = yb * w_ref[:, 0:1]


def _expert_of_tile(k, offs):
    ex = 0
    for j in range(1, E + 1):
        ex = ex + jnp.where(offs[j] <= k * BT, 1, 0)
    return jnp.minimum(ex, E - 1)


def _ffn_call(offs16, xs, ws, W1, b1, W2, b2):
    grid_spec = pltpu.PrefetchScalarGridSpec(
        num_scalar_prefetch=1,
        grid=(NTP,),
        in_specs=[
            pl.BlockSpec((BT, D), lambda k, offs: (k, 0)),
            pl.BlockSpec((BT, LANES), lambda k, offs: (k, 0)),
            pl.BlockSpec((1, D, F), lambda k, offs: (_expert_of_tile(k, offs), 0, 0)),
            pl.BlockSpec((1, 1, F), lambda k, offs: (_expert_of_tile(k, offs), 0, 0)),
            pl.BlockSpec((1, F, D), lambda k, offs: (_expert_of_tile(k, offs), 0, 0)),
            pl.BlockSpec((1, 1, D), lambda k, offs: (_expert_of_tile(k, offs), 0, 0)),
        ],
        out_specs=pl.BlockSpec((BT, D), lambda k, offs: (k, 0)),
    )
    return pl.pallas_call(
        _ffn_body,
        grid_spec=grid_spec,
        out_shape=jax.ShapeDtypeStruct((TP, D), jnp.float32),
        compiler_params=pltpu.CompilerParams(
            dimension_semantics=("arbitrary",)),
    )(offs16, xs, ws, W1, b1.reshape(E, 1, F), W2, b2.reshape(E, 1, D))


def kernel(x, Wr, W1, b1, W2, b2):
    batch, seq, _ = x.shape
    x_flat = x.reshape(T, D)

    destrep, wrep, offs16, aux1 = _router_call(x_flat, Wr)
    dest1d = destrep[:, 0]

    xs, ws = _dispatch_call(dest1d, x_flat, wrep)
    y_sorted = _ffn_call(offs16, xs, ws, W1, b1, W2, b2)
    out_flat = _combine_call(dest1d, y_sorted)

    return out_flat.reshape(batch, seq, D), aux1[0]


# speedup vs baseline: 1.2553x; 1.2553x over previous
import functools

import jax
import jax.numpy as jnp
from jax import lax
from jax.experimental import pallas as pl
from jax.experimental.pallas import tpu as pltpu
from jax.experimental.pallas import tpu_sc as plsc

D = 768
E = 8
F = 3072
T = 2048
LANES = 128
BT = 256
TP = T + E * BT
NTP = TP // BT
LBW = 0.01

NC = 2
NS = 16
NW = NC * NS
BPW = T // NW


def _router_body(x_ref, wr_ref, dest_ref, wrep_ref, offs_ref, aux_ref):
    x = x_ref[...]
    wr = wr_ref[...]
    logits = jnp.dot(x, wr, preferred_element_type=jnp.float32)
    m = jnp.max(logits, axis=1, keepdims=True)
    u = jnp.exp(logits - m)
    s = jnp.sum(u, axis=1, keepdims=True)
    probs = u / s
    wmax = jnp.max(probs, axis=1, keepdims=True)
    lane8 = lax.broadcasted_iota(jnp.int32, (T, E), 1)
    idx = jnp.min(jnp.where(probs == wmax, lane8, E), axis=1,
                  keepdims=True)
    lane = lax.broadcasted_iota(jnp.int32, (T, LANES), 1)
    onehot = (lane == idx).astype(jnp.float32)

    c = onehot
    sh = 1
    while sh < T:
        c = c + jnp.concatenate(
            [jnp.zeros((sh, LANES), jnp.float32), c[:-sh]], axis=0)
        sh *= 2
    counts = c[T - 1:T, :]

    i2 = lax.broadcasted_iota(jnp.int32, (LANES, LANES), 0)
    j2 = lax.broadcasted_iota(jnp.int32, (LANES, LANES), 1)
    tri = (i2 < j2).astype(jnp.float32)
    padded = jnp.floor((counts + (BT - 1)) / BT) * BT
    counts8 = jnp.broadcast_to(padded, (8, LANES))
    offs8 = jnp.dot(counts8, tri, preferred_element_type=jnp.float32,
                    precision=lax.Precision.HIGHEST)
    offs = offs8[0:1, :]

    dest = jnp.sum(onehot * (offs + c - 1.0), axis=1, keepdims=True)
    dest_ref[...] = jnp.broadcast_to(dest, (T, LANES)).astype(jnp.int32)
    wrep_ref[...] = jnp.broadcast_to(wmax, (T, LANES))

    lane_row = lane[0:1, :]
    for i in range(16):
        oi = jnp.sum(jnp.where(lane_row == i, offs, 0.0))
        offs_ref[i] = oi.astype(jnp.int32)

    mean_p8 = jnp.sum(probs, axis=0, keepdims=True) / T
    counts8 = jnp.sum((lane8 == idx).astype(jnp.float32), axis=0,
                      keepdims=True)
    aux_ref[0] = LBW * E * jnp.sum(mean_p8 * (counts8 / T))


def _router_call(x_flat, wr):
    return pl.pallas_call(
        _router_body,
        out_shape=(
            jax.ShapeDtypeStruct((T, LANES), jnp.int32),
            jax.ShapeDtypeStruct((T, LANES), jnp.float32),
            jax.ShapeDtypeStruct((16,), jnp.int32),
            jax.ShapeDtypeStruct((1,), jnp.float32),
        ),
        out_specs=(
            pl.BlockSpec(memory_space=pltpu.MemorySpace.VMEM),
            pl.BlockSpec(memory_space=pltpu.MemorySpace.VMEM),
            pl.BlockSpec(memory_space=pltpu.SMEM),
            pl.BlockSpec(memory_space=pltpu.SMEM),
        ),
    )(x_flat, wr)


def _dispatch_call(dest1d, x_flat, wrep):
    mesh = plsc.VectorSubcoreMesh(core_axis_name="c", subcore_axis_name="s")

    def body(dest_hbm, x_hbm, w_hbm, xs_hbm, ws_hbm, idx_v, xrows_v, wrows_v,
             sem1, sem2):
        wid = lax.axis_index("s") * NC + lax.axis_index("c")
        base = wid * BPW
        pltpu.sync_copy(dest_hbm.at[pl.ds(base, BPW)], idx_v)
        pltpu.sync_copy(x_hbm.at[pl.ds(base, BPW)], xrows_v)
        pltpu.sync_copy(w_hbm.at[pl.ds(base, BPW)], wrows_v)
        pltpu.async_copy(xrows_v, xs_hbm.at[idx_v], sem1).wait()
        pltpu.async_copy(wrows_v, ws_hbm.at[idx_v], sem2).wait()

    f = pl.kernel(
        body,
        out_type=(
            jax.ShapeDtypeStruct((TP, D), jnp.float32),
            jax.ShapeDtypeStruct((TP, LANES), jnp.float32),
        ),
        mesh=mesh,
        scratch_types=[
            pltpu.VMEM((BPW,), jnp.int32),
            pltpu.VMEM((BPW, D), jnp.float32),
            pltpu.VMEM((BPW, LANES), jnp.float32),
            pltpu.SemaphoreType.DMA,
            pltpu.SemaphoreType.DMA,
        ],
    )
    return f(dest1d, x_flat, wrep)


def _combine_call(dest1d, y_sorted):
    mesh = plsc.VectorSubcoreMesh(core_axis_name="c", subcore_axis_name="s")

    def body(dest_hbm, y_hbm, out_hbm, idx_v, rows_v, sem):
        wid = lax.axis_index("s") * NC + lax.axis_index("c")
        base = wid * BPW
        pltpu.sync_copy(dest_hbm.at[pl.ds(base, BPW)], idx_v)
        pltpu.async_copy(y_hbm.at[idx_v], rows_v, sem).wait()
        pltpu.sync_copy(rows_v, out_hbm.at[pl.ds(base, BPW)])

    f = pl.kernel(
        body,
        out_type=jax.ShapeDtypeStruct((T, D), jnp.float32),
        mesh=mesh,
        scratch_types=[
            pltpu.VMEM((BPW,), jnp.int32),
            pltpu.VMEM((BPW, D), jnp.float32),
            pltpu.SemaphoreType.DMA,
        ],
    )
    return f(dest1d, y_sorted)


_SQRT_HALF = 0.7071067811865476


def _ffn_body(offs_sref, x_ref, w_ref, w1_ref, b1_ref, w2_ref, b2_ref, out_ref):
    xb = x_ref[...]
    h = jnp.dot(xb, w1_ref[0], preferred_element_type=jnp.float32)
    h = h + b1_ref[0]
    g = 0.5 * h * (1.0 + lax.erf(h * _SQRT_HALF))
    yb = jnp.dot(g, w2_ref[0], preferred_element_type=jnp.float32)
    yb = yb + b2_ref[0]
    out_ref[...] = yb * w_ref[:, 0:1]


def _expert_of_tile(k, offs):
    ex = 0
    for j in range(1, E + 1):
        ex = ex + jnp.where(offs[j] <= k * BT, 1, 0)
    return jnp.minimum(ex, E - 1)


def _ffn_call(offs16, xs, ws, W1, b1, W2, b2):
    grid_spec = pltpu.PrefetchScalarGridSpec(
        num_scalar_prefetch=1,
        grid=(NTP,),
        in_specs=[
            pl.BlockSpec((BT, D), lambda k, offs: (k, 0)),
            pl.BlockSpec((BT, LANES), lambda k, offs: (k, 0)),
            pl.BlockSpec((1, D, F), lambda k, offs: (_expert_of_tile(k, offs), 0, 0)),
            pl.BlockSpec((1, 1, F), lambda k, offs: (_expert_of_tile(k, offs), 0, 0)),
            pl.BlockSpec((1, F, D), lambda k, offs: (_expert_of_tile(k, offs), 0, 0)),
            pl.BlockSpec((1, 1, D), lambda k, offs: (_expert_of_tile(k, offs), 0, 0)),
        ],
        out_specs=pl.BlockSpec((BT, D), lambda k, offs: (k, 0)),
    )
    return pl.pallas_call(
        _ffn_body,
        grid_spec=grid_spec,
        out_shape=jax.ShapeDtypeStruct((TP, D), jnp.float32),
        compiler_params=pltpu.CompilerParams(
            dimension_semantics=("arbitrary",)),
    )(offs16, xs, ws, W1, b1.reshape(E, 1, F), W2, b2.reshape(E, 1, D))


def kernel(x, Wr, W1, b1, W2, b2):
    batch, seq, _ = x.shape
    x_flat = x.reshape(T, D)

    destrep, wrep, offs16, aux1 = _router_call(x_flat, Wr)
    dest1d = destrep[:, 0]

    xs, ws = _dispatch_call(dest1d, x_flat, wrep)
    y_sorted = _ffn_call(offs16, xs, ws, W1, b1, W2, b2)
    out_flat = _combine_call(dest1d, y_sorted)

    return out_flat.reshape(batch, seq, D), aux1[0]


# speedup vs baseline: 1.3232x; 1.0540x over previous
import functools

import jax
import jax.numpy as jnp
from jax import lax
from jax.experimental import pallas as pl
from jax.experimental.pallas import tpu as pltpu
from jax.experimental.pallas import tpu_sc as plsc

D = 768
E = 8
F = 3072
T = 2048
LANES = 128
BT = 256
TP = T + E * BT
NTP = TP // BT
LBW = 0.01

NC = 2
NS = 16
NW = NC * NS
BPW = T // NW


def _router_body(x_ref, wr_ref, dest_ref, wrep_ref, offs_ref, aux_ref):
    x = x_ref[...]
    wr = wr_ref[...]
    logits = jnp.dot(x, wr, preferred_element_type=jnp.float32)
    m = jnp.max(logits, axis=1, keepdims=True)
    u = jnp.exp(logits - m)
    s = jnp.sum(u, axis=1, keepdims=True)
    probs = u / s
    wmax = jnp.max(probs, axis=1, keepdims=True)
    lane8 = lax.broadcasted_iota(jnp.int32, (T, E), 1)
    idx = jnp.min(jnp.where(probs == wmax, lane8, E), axis=1,
                  keepdims=True)
    lane = lax.broadcasted_iota(jnp.int32, (T, LANES), 1)
    onehot = (lane == idx).astype(jnp.float32)

    c = onehot
    sh = 1
    while sh < T:
        c = c + jnp.concatenate(
            [jnp.zeros((sh, LANES), jnp.float32), c[:-sh]], axis=0)
        sh *= 2
    counts = c[T - 1:T, :]

    i2 = lax.broadcasted_iota(jnp.int32, (LANES, LANES), 0)
    j2 = lax.broadcasted_iota(jnp.int32, (LANES, LANES), 1)
    tri = (i2 < j2).astype(jnp.float32)
    padded = jnp.floor((counts + (BT - 1)) / BT) * BT
    counts8 = jnp.broadcast_to(padded, (8, LANES))
    offs8 = jnp.dot(counts8, tri, preferred_element_type=jnp.float32,
                    precision=lax.Precision.HIGHEST)
    offs = offs8[0:1, :]

    dest = jnp.sum(onehot * (offs + c - 1.0), axis=1, keepdims=True)
    dest_ref[...] = jnp.broadcast_to(dest, (T, LANES)).astype(jnp.int32)
    wrep_ref[...] = jnp.broadcast_to(wmax, (T, LANES))

    lane_row = lane[0:1, :]
    for i in range(16):
        oi = jnp.sum(jnp.where(lane_row == i, offs, 0.0))
        offs_ref[i] = oi.astype(jnp.int32)

    mean_p8 = jnp.sum(probs, axis=0, keepdims=True) / T
    counts8 = jnp.sum((lane8 == idx).astype(jnp.float32), axis=0,
                      keepdims=True)
    aux_ref[0] = LBW * E * jnp.sum(mean_p8 * (counts8 / T))


def _router_call(x_flat, wr):
    return pl.pallas_call(
        _router_body,
        out_shape=(
            jax.ShapeDtypeStruct((T, LANES), jnp.int32),
            jax.ShapeDtypeStruct((T, LANES), jnp.float32),
            jax.ShapeDtypeStruct((16,), jnp.int32),
            jax.ShapeDtypeStruct((1,), jnp.float32),
        ),
        out_specs=(
            pl.BlockSpec(memory_space=pltpu.MemorySpace.VMEM),
            pl.BlockSpec(memory_space=pltpu.MemorySpace.VMEM),
            pl.BlockSpec(memory_space=pltpu.SMEM),
            pl.BlockSpec(memory_space=pltpu.SMEM),
        ),
    )(x_flat, wr)


def _dispatch_call(dest1d, x_flat, wrep):
    mesh = plsc.VectorSubcoreMesh(core_axis_name="c", subcore_axis_name="s")

    def body(dest_hbm, x_hbm, w_hbm, xs_hbm, ws_hbm, idx_v, xrows_v, wrows_v,
             sem1, sem2):
        wid = lax.axis_index("s") * NC + lax.axis_index("c")
        base = wid * BPW
        pltpu.sync_copy(dest_hbm.at[pl.ds(base, BPW)], idx_v)
        pltpu.sync_copy(x_hbm.at[pl.ds(base, BPW)], xrows_v)
        pltpu.sync_copy(w_hbm.at[pl.ds(base, BPW)], wrows_v)
        pltpu.async_copy(xrows_v, xs_hbm.at[idx_v], sem1).wait()
        pltpu.async_copy(wrows_v, ws_hbm.at[idx_v], sem2).wait()

    f = pl.kernel(
        body,
        out_type=(
            jax.ShapeDtypeStruct((TP, D), jnp.float32),
            jax.ShapeDtypeStruct((TP, LANES), jnp.float32),
        ),
        mesh=mesh,
        scratch_types=[
            pltpu.VMEM((BPW,), jnp.int32),
            pltpu.VMEM((BPW, D), jnp.float32),
            pltpu.VMEM((BPW, LANES), jnp.float32),
            pltpu.SemaphoreType.DMA,
            pltpu.SemaphoreType.DMA,
        ],
    )
    return f(dest1d, x_flat, wrep)


def _combine_call(dest1d, y_sorted):
    mesh = plsc.VectorSubcoreMesh(core_axis_name="c", subcore_axis_name="s")

    def body(dest_hbm, y_hbm, out_hbm, idx_v, rows_v, sem):
        wid = lax.axis_index("s") * NC + lax.axis_index("c")
        base = wid * BPW
        pltpu.sync_copy(dest_hbm.at[pl.ds(base, BPW)], idx_v)
        pltpu.async_copy(y_hbm.at[idx_v], rows_v, sem).wait()
        pltpu.sync_copy(rows_v, out_hbm.at[pl.ds(base, BPW)])

    f = pl.kernel(
        body,
        out_type=jax.ShapeDtypeStruct((T, D), jnp.float32),
        mesh=mesh,
        scratch_types=[
            pltpu.VMEM((BPW,), jnp.int32),
            pltpu.VMEM((BPW, D), jnp.float32),
            pltpu.SemaphoreType.DMA,
        ],
    )
    return f(dest1d, y_sorted)


_SQRT_HALF = 0.7071067811865476


def _ffn_body(offs_sref, x_ref, w_ref, w1_ref, b1_ref, w2_ref, b2_ref, out_ref):
    k = pl.program_id(0)

    @pl.when(k * BT < offs_sref[E])
    def _compute():
        xb = x_ref[...]
        h = jnp.dot(xb, w1_ref[0], preferred_element_type=jnp.float32)
        h = h + b1_ref[0]
        g = 0.5 * h * (1.0 + lax.erf(h * _SQRT_HALF))
        yb = jnp.dot(g, w2_ref[0], preferred_element_type=jnp.float32)
        yb = yb + b2_ref[0]
        out_ref[...] = yb * w_ref[:, 0:1]


def _expert_of_tile(k, offs):
    ex = 0
    for j in range(1, E + 1):
        ex = ex + jnp.where(offs[j] <= k * BT, 1, 0)
    return jnp.minimum(ex, E - 1)


def _ffn_call(offs16, xs, ws, W1, b1, W2, b2):
    grid_spec = pltpu.PrefetchScalarGridSpec(
        num_scalar_prefetch=1,
        grid=(NTP,),
        in_specs=[
            pl.BlockSpec((BT, D), lambda k, offs: (k, 0)),
            pl.BlockSpec((BT, LANES), lambda k, offs: (k, 0)),
            pl.BlockSpec((1, D, F), lambda k, offs: (_expert_of_tile(k, offs), 0, 0)),
            pl.BlockSpec((1, 1, F), lambda k, offs: (_expert_of_tile(k, offs), 0, 0)),
            pl.BlockSpec((1, F, D), lambda k, offs: (_expert_of_tile(k, offs), 0, 0)),
            pl.BlockSpec((1, 1, D), lambda k, offs: (_expert_of_tile(k, offs), 0, 0)),
        ],
        out_specs=pl.BlockSpec((BT, D), lambda k, offs: (k, 0)),
    )
    return pl.pallas_call(
        _ffn_body,
        grid_spec=grid_spec,
        out_shape=jax.ShapeDtypeStruct((TP, D), jnp.float32),
        compiler_params=pltpu.CompilerParams(
            dimension_semantics=("arbitrary",)),
    )(offs16, xs, ws, W1, b1.reshape(E, 1, F), W2, b2.reshape(E, 1, D))


def kernel(x, Wr, W1, b1, W2, b2):
    batch, seq, _ = x.shape
    x_flat = x.reshape(T, D)

    destrep, wrep, offs16, aux1 = _router_call(x_flat, Wr)
    dest1d = destrep[:, 0]

    xs, ws = _dispatch_call(dest1d, x_flat, wrep)
    y_sorted = _ffn_call(offs16, xs, ws, W1, b1, W2, b2)
    out_flat = _combine_call(dest1d, y_sorted)

    return out_flat.reshape(batch, seq, D), aux1[0]


# speedup vs baseline: 1.3399x; 1.0126x over previous
import functools

import jax
import jax.numpy as jnp
from jax import lax
from jax.experimental import pallas as pl
from jax.experimental.pallas import tpu as pltpu
from jax.experimental.pallas import tpu_sc as plsc

D = 768
E = 8
F = 3072
T = 2048
LANES = 128
BT = 256
TP = T + E * BT
NTP = TP // BT
LBW = 0.01

NC = 2
NS = 16
NW = NC * NS
BPW = T // NW


def _router_body(x_ref, wr_ref, dest_ref, wrep_ref, offs_ref, aux_ref):
    x = x_ref[...]
    wr = wr_ref[...]
    logits = jnp.dot(x, wr, preferred_element_type=jnp.float32)
    m = jnp.max(logits, axis=1, keepdims=True)
    u = jnp.exp(logits - m)
    s = jnp.sum(u, axis=1, keepdims=True)
    probs = u / s
    wmax = jnp.max(probs, axis=1, keepdims=True)
    lane8 = lax.broadcasted_iota(jnp.int32, (T, E), 1)
    idx = jnp.min(jnp.where(probs == wmax, lane8, E), axis=1,
                  keepdims=True)
    lane = lax.broadcasted_iota(jnp.int32, (T, LANES), 1)
    onehot = (lane == idx).astype(jnp.float32)

    c = onehot
    sh = 1
    while sh < T:
        c = c + jnp.concatenate(
            [jnp.zeros((sh, LANES), jnp.float32), c[:-sh]], axis=0)
        sh *= 2
    counts = c[T - 1:T, :]

    i2 = lax.broadcasted_iota(jnp.int32, (LANES, LANES), 0)
    j2 = lax.broadcasted_iota(jnp.int32, (LANES, LANES), 1)
    tri = (i2 < j2).astype(jnp.float32)
    padded = jnp.floor((counts + (BT - 1)) / BT) * BT
    counts8 = jnp.broadcast_to(padded, (8, LANES))
    offs8 = jnp.dot(counts8, tri, preferred_element_type=jnp.float32,
                    precision=lax.Precision.HIGHEST)
    offs = offs8[0:1, :]

    dest = jnp.sum(onehot * (offs + c - 1.0), axis=1, keepdims=True)
    dest_ref[...] = jnp.broadcast_to(dest, (T, LANES)).astype(jnp.int32)
    wrep_ref[...] = jnp.broadcast_to(wmax, (T, LANES))

    lane_row = lane[0:1, :]
    for i in range(16):
        oi = jnp.sum(jnp.where(lane_row == i, offs, 0.0))
        offs_ref[i] = oi.astype(jnp.int32)

    mean_p8 = jnp.sum(probs, axis=0, keepdims=True) / T
    counts8 = jnp.sum((lane8 == idx).astype(jnp.float32), axis=0,
                      keepdims=True)
    aux_ref[0] = LBW * E * jnp.sum(mean_p8 * (counts8 / T))


def _router_call(x_flat, wr):
    return pl.pallas_call(
        _router_body,
        out_shape=(
            jax.ShapeDtypeStruct((T, LANES), jnp.int32),
            jax.ShapeDtypeStruct((T, LANES), jnp.float32),
            jax.ShapeDtypeStruct((16,), jnp.int32),
            jax.ShapeDtypeStruct((1,), jnp.float32),
        ),
        out_specs=(
            pl.BlockSpec(memory_space=pltpu.MemorySpace.VMEM),
            pl.BlockSpec(memory_space=pltpu.MemorySpace.VMEM),
            pl.BlockSpec(memory_space=pltpu.SMEM),
            pl.BlockSpec(memory_space=pltpu.SMEM),
        ),
    )(x_flat, wr)


def _dispatch_call(dest1d, x_flat, wrep):
    mesh = plsc.VectorSubcoreMesh(core_axis_name="c", subcore_axis_name="s")

    def body(dest_hbm, x_hbm, w_hbm, xs_hbm, ws_hbm, idx_v, xrows_v, wrows_v,
             sem1, sem2, sem3, sem4):
        wid = lax.axis_index("s") * NC + lax.axis_index("c")
        base = wid * BPW
        a1 = pltpu.async_copy(x_hbm.at[pl.ds(base, BPW)], xrows_v, sem3)
        a2 = pltpu.async_copy(w_hbm.at[pl.ds(base, BPW)], wrows_v, sem4)
        pltpu.sync_copy(dest_hbm.at[pl.ds(base, BPW)], idx_v)
        a1.wait()
        c1 = pltpu.async_copy(xrows_v, xs_hbm.at[idx_v], sem1)
        a2.wait()
        c2 = pltpu.async_copy(wrows_v, ws_hbm.at[idx_v], sem2)
        c1.wait()
        c2.wait()

    f = pl.kernel(
        body,
        out_type=(
            jax.ShapeDtypeStruct((TP, D), jnp.float32),
            jax.ShapeDtypeStruct((TP, LANES), jnp.float32),
        ),
        mesh=mesh,
        scratch_types=[
            pltpu.VMEM((BPW,), jnp.int32),
            pltpu.VMEM((BPW, D), jnp.float32),
            pltpu.VMEM((BPW, LANES), jnp.float32),
            pltpu.SemaphoreType.DMA,
            pltpu.SemaphoreType.DMA,
            pltpu.SemaphoreType.DMA,
            pltpu.SemaphoreType.DMA,
        ],
    )
    return f(dest1d, x_flat, wrep)


def _combine_call(dest1d, y_sorted):
    mesh = plsc.VectorSubcoreMesh(core_axis_name="c", subcore_axis_name="s")

    def body(dest_hbm, y_hbm, out_hbm, idx_v, rows_v, sem):
        wid = lax.axis_index("s") * NC + lax.axis_index("c")
        base = wid * BPW
        pltpu.sync_copy(dest_hbm.at[pl.ds(base, BPW)], idx_v)
        pltpu.async_copy(y_hbm.at[idx_v], rows_v, sem).wait()
        pltpu.sync_copy(rows_v, out_hbm.at[pl.ds(base, BPW)])

    f = pl.kernel(
        body,
        out_type=jax.ShapeDtypeStruct((T, D), jnp.float32),
        mesh=mesh,
        scratch_types=[
            pltpu.VMEM((BPW,), jnp.int32),
            pltpu.VMEM((BPW, D), jnp.float32),
            pltpu.SemaphoreType.DMA,
        ],
    )
    return f(dest1d, y_sorted)


_SQRT_HALF = 0.7071067811865476


def _ffn_body(offs_sref, x_ref, w_ref, w1_ref, b1_ref, w2_ref, b2_ref, out_ref):
    k = pl.program_id(0)

    @pl.when(k * BT < offs_sref[E])
    def _compute():
        xb = x_ref[...]
        h = jnp.dot(xb, w1_ref[0], preferred_element_type=jnp.float32)
        h = h + b1_ref[0]
        g = 0.5 * h * (1.0 + lax.erf(h * _SQRT_HALF))
        yb = jnp.dot(g, w2_ref[0], preferred_element_type=jnp.float32)
        yb = yb + b2_ref[0]
        out_ref[...] = yb * w_ref[:, 0:1]


def _expert_of_tile(k, offs):
    ex = 0
    for j in range(1, E + 1):
        ex = ex + jnp.where(offs[j] <= k * BT, 1, 0)
    return jnp.minimum(ex, E - 1)


def _ffn_call(offs16, xs, ws, W1, b1, W2, b2):
    grid_spec = pltpu.PrefetchScalarGridSpec(
        num_scalar_prefetch=1,
        grid=(NTP,),
        in_specs=[
            pl.BlockSpec((BT, D), lambda k, offs: (k, 0)),
            pl.BlockSpec((BT, LANES), lambda k, offs: (k, 0)),
            pl.BlockSpec((1, D, F), lambda k, offs: (_expert_of_tile(k, offs), 0, 0)),
            pl.BlockSpec((1, 1, F), lambda k, offs: (_expert_of_tile(k, offs), 0, 0)),
            pl.BlockSpec((1, F, D), lambda k, offs: (_expert_of_tile(k, offs), 0, 0)),
            pl.BlockSpec((1, 1, D), lambda k, offs: (_expert_of_tile(k, offs), 0, 0)),
        ],
        out_specs=pl.BlockSpec((BT, D), lambda k, offs: (k, 0)),
    )
    return pl.pallas_call(
        _ffn_body,
        grid_spec=grid_spec,
        out_shape=jax.ShapeDtypeStruct((TP, D), jnp.float32),
        compiler_params=pltpu.CompilerParams(
            dimension_semantics=("arbitrary",)),
    )(offs16, xs, ws, W1, b1.reshape(E, 1, F), W2, b2.reshape(E, 1, D))


def kernel(x, Wr, W1, b1, W2, b2):
    batch, seq, _ = x.shape
    x_flat = x.reshape(T, D)

    destrep, wrep, offs16, aux1 = _router_call(x_flat, Wr)
    dest1d = destrep[:, 0]

    xs, ws = _dispatch_call(dest1d, x_flat, wrep)
    y_sorted = _ffn_call(offs16, xs, ws, W1, b1, W2, b2)
    out_flat = _combine_call(dest1d, y_sorted)

    return out_flat.reshape(batch, seq, D), aux1[0]
